# baseline (speedup 1.0000x reference)
import jax
import jax.numpy as jnp
from jax.experimental import pallas as pl
from jax.experimental.pallas import tpu as pltpu

_N = 10000
_F_IN = 128
_HID = 128
_NCLASS = 64
_BM = 400
_BH = _BM // 2
_NBLK = _N // _BM


def _gcn_body(x_ref, w1_ref, adj_a_ref, adj_b_ref, mask_ref, b1_ref,
              w2_ref, b2_ref, o_ref, s1_ref, s2_ref):
    i = pl.program_id(0)

    @pl.when(i == 0)
    def _():
        s1_ref[...] = jnp.dot(x_ref[...], w1_ref[...],
                              preferred_element_type=jnp.float32)

    @pl.when(i < _NBLK)
    def _():
        s_top = jnp.dot(adj_a_ref[...], s1_ref[...],
                        preferred_element_type=jnp.float32)
        s_bot = jnp.dot(adj_b_ref[...], s1_ref[...],
                        preferred_element_type=jnp.float32)
        h_top = (s_top + b1_ref[...]) * mask_ref[:_BH, :]
        h_bot = (s_bot + b1_ref[...]) * mask_ref[_BH:, :]
        s2_ref[pl.ds(i * _BM, _BH), :] = jnp.dot(
            h_top, w2_ref[...], preferred_element_type=jnp.float32)
        s2_ref[pl.ds(i * _BM + _BH, _BH), :] = jnp.dot(
            h_bot, w2_ref[...], preferred_element_type=jnp.float32)

    @pl.when(i >= _NBLK)
    def _():
        lg_top = jnp.dot(adj_a_ref[...], s2_ref[...],
                         preferred_element_type=jnp.float32) + b2_ref[...]
        lg_bot = jnp.dot(adj_b_ref[...], s2_ref[...],
                         preferred_element_type=jnp.float32) + b2_ref[...]
        mx_t = jnp.max(lg_top, axis=1, keepdims=True)
        mx_b = jnp.max(lg_bot, axis=1, keepdims=True)
        lse_t = jnp.log(jnp.sum(jnp.exp(lg_top - mx_t), axis=1,
                                keepdims=True)) + mx_t
        lse_b = jnp.log(jnp.sum(jnp.exp(lg_bot - mx_b), axis=1,
                                keepdims=True)) + mx_b
        o_ref[:_BH, :] = lg_top - lse_t
        o_ref[_BH:, :] = lg_bot - lse_b


def kernel(x, adj, W1, b1, W2, b2):
    keep = jax.random.bernoulli(jax.random.key(42), 0.5, (_N, _HID))
    maskf = keep.astype(jnp.float32) * 2.0
    b1r = b1.reshape(1, _HID)
    b2r = b2.reshape(1, _NCLASS)

    out = pl.pallas_call(
        _gcn_body,
        grid=(2 * _NBLK,),
        in_specs=[
            pl.BlockSpec((_N, _F_IN), lambda i: (0, 0)),
            pl.BlockSpec((_F_IN, _HID), lambda i: (0, 0)),
            pl.BlockSpec((_BH, _N),
                         lambda i: (2 * (i % _NBLK), 0)),
            pl.BlockSpec((_BH, _N),
                         lambda i: (2 * (i % _NBLK) + 1, 0)),
            pl.BlockSpec((_BM, _HID),
                         lambda i: (jnp.minimum(i, _NBLK - 1), 0)),
            pl.BlockSpec((1, _HID), lambda i: (0, 0)),
            pl.BlockSpec((_HID, _NCLASS), lambda i: (0, 0)),
            pl.BlockSpec((1, _NCLASS), lambda i: (0, 0)),
        ],
        out_specs=pl.BlockSpec(
            (_BM, _NCLASS),
            lambda i: (jnp.where(i < _NBLK, 0, i - _NBLK), 0)),
        out_shape=jax.ShapeDtypeStruct((_N, _NCLASS), jnp.float32),
        scratch_shapes=[
            pltpu.VMEM((_N, _HID), jnp.float32),
            pltpu.VMEM((_N, _NCLASS), jnp.float32),
        ],
        compiler_params=pltpu.CompilerParams(
            dimension_semantics=("arbitrary",)),
    )(x, W1, adj, adj, maskf, b1r, W2, b2r)

    return out
